# baseline (speedup 1.0000x reference)
import functools

import jax
import jax.numpy as jnp
from jax import lax
from jax.experimental import pallas as pl
from jax.experimental.pallas import tpu as pltpu
from jax.experimental.pallas import tpu_sc as plsc

NC = 2
NS = 16
NW = NC * NS

S = 128
G = 8


@functools.partial(jax.jit, static_argnames=("n_slices", "d"))
def _sc_gather(flat_idx, flat_table, *, n_slices, d):
    slices_per_w = n_slices // NW
    n_chunks = slices_per_w // G

    mesh = plsc.VectorSubcoreMesh(core_axis_name="c", subcore_axis_name="s")

    @functools.partial(
        pl.kernel,
        out_type=jax.ShapeDtypeStruct((n_slices, S, d), jnp.float32),
        mesh=mesh,
        compiler_params=pltpu.CompilerParams(use_tc_tiling_on_sc=False),
        scratch_types=[
            pltpu.VMEM((G, S), jnp.int32),
            pltpu.VMEM((G, S, d), jnp.float32),
            pltpu.SemaphoreType.DMA,
        ],
    )
    def gather_kernel(idx_hbm, table_hbm, out_hbm, idx_v, rows_v, sem):
        wid = lax.axis_index("s") * NC + lax.axis_index("c")
        base = wid * slices_per_w

        def chunk_body(ci, carry):
            row0 = base + ci * G
            pltpu.sync_copy(idx_hbm.at[pl.ds(row0, G)], idx_v)
            copies = [
                pltpu.async_copy(table_hbm.at[idx_v.at[j]], rows_v.at[j], sem)
                for j in range(G)
            ]
            for c in copies:
                c.wait()
            pltpu.sync_copy(rows_v, out_hbm.at[pl.ds(row0, G)])
            return carry

        lax.fori_loop(0, n_chunks, chunk_body, 0)

    return gather_kernel(flat_idx, flat_table)


def kernel(inputs, tables):
    f, v, d = tables.shape
    b = inputs.shape[0]
    n_rows = b * f
    n_slices = n_rows // S

    offsets = (jnp.arange(f, dtype=jnp.int32) * v)[None, :]
    flat_idx = (inputs + offsets).reshape(n_slices, S)
    flat_table = tables.reshape(f * v, d)

    out = _sc_gather(flat_idx, flat_table, n_slices=n_slices, d=d)
    return out.reshape(b, f, d)
